# baseline (device time: 114735 ns/iter reference)
import jax
import jax.numpy as jnp
from jax import lax
from jax.experimental import pallas as pl
from jax.experimental.pallas import tpu as pltpu

N_DEV = 16
B, SQ, SKV, DH = 2, 512, 512, 64
HQ_LOCAL = 8
D_LOCAL = HQ_LOCAL * DH
D_MODEL = 768
ROWS = B * SQ
CHUNK = ROWS // N_DEV
NG = 4
GROUP = 2 * 64

_MESH = pl.DeviceIdType.MESH

_PERM = [0, 4, 1, 5, 2, 6, 3, 7]


def _permute_rows(a):
    return jnp.concatenate([a[p * 64:(p + 1) * 64] for p in _PERM], axis=0)


def _body(x_ref, wq_ref, k_ref, v_ref, wo_ref, out_ref,
          snd_ref, ctx_ref, a2a_ref, red_ref, flat_ref,
          s1_send, s1_recv, s2_send, s2_recv):
    my = lax.axis_index("i")

    barrier_sem = pltpu.get_barrier_semaphore()
    for dj in range(1, N_DEV):
        peer = lax.rem(my + dj, N_DEV)
        pl.semaphore_signal(barrier_sem, inc=1, device_id=(peer,),
                            device_id_type=_MESH)
    pl.semaphore_wait(barrier_sem, N_DEV - 1)

    dests = {}
    rdmas1 = {}
    for dj in range(1, N_DEV):
        d = lax.rem(my + dj, N_DEV)
        dests[dj] = d
        rdmas1[dj] = pltpu.make_async_remote_copy(
            src_ref=snd_ref.at[pl.ds(d * CHUNK, CHUNK), :],
            dst_ref=a2a_ref.at[dj - 1],
            send_sem=s1_send.at[dj - 1],
            recv_sem=s1_recv.at[dj - 1],
            device_id=(d,),
            device_id_type=_MESH,
        )

    for b in range(B):
        q_b = jnp.dot(x_ref[b], wq_ref[...],
                      preferred_element_type=jnp.float32).astype(jnp.bfloat16)
        q_perm = _permute_rows(q_b)
        k_perm = _permute_rows(k_ref[b])
        v_perm = _permute_rows(v_ref[b])
        for h in range(HQ_LOCAL):
            for g in range(NG):
                r = slice(g * GROUP, (g + 1) * GROUP)
                qg = q_perm[r, h * DH:(h + 1) * DH]
                kg = k_perm[r, h, :]
                s = lax.dot_general(qg, kg, (((1,), (1,)), ((), ())),
                                    preferred_element_type=jnp.float32)
                w = jnp.exp(s * 0.125)
                w = (w / jnp.sum(w, axis=1, keepdims=True)).astype(jnp.bfloat16)
                ctx = jnp.dot(w, v_perm[r, h, :],
                              preferred_element_type=jnp.float32)
                ctx = ctx.astype(jnp.bfloat16)
                hc = slice(h * DH, (h + 1) * DH)
                ctx_ref[b, g * 64:(g + 1) * 64, hc] = ctx[0:64]
                ctx_ref[b, (g + 4) * 64:(g + 5) * 64, hc] = ctx[64:128]
        proj = jnp.dot(ctx_ref[b], wo_ref[...],
                       preferred_element_type=jnp.float32)
        snd_ref[b * SQ:(b + 1) * SQ, :] = proj.astype(jnp.bfloat16)

        for dj in range(1, N_DEV):
            ready = (dests[dj] < 8) if b == 0 else (dests[dj] >= 8)
            rdma = rdmas1[dj]

            @pl.when(ready)
            def _(rdma=rdma):
                rdma.start()

    red = snd_ref[pl.ds(my * CHUNK, CHUNK), :].astype(jnp.float32)
    for k in range(N_DEV - 1):
        recv = pltpu.make_async_remote_copy(
            src_ref=a2a_ref.at[k], dst_ref=a2a_ref.at[k],
            send_sem=s1_send.at[k], recv_sem=s1_recv.at[k],
            device_id=(my,), device_id_type=_MESH,
        )
        recv.wait_recv()
        red = red + a2a_ref[k].astype(jnp.float32)
    red_ref[...] = red.astype(jnp.bfloat16)
    flat_ref[pl.ds(my * CHUNK, CHUNK), :] = red_ref[...]
    for dj in range(1, N_DEV):
        rdmas1[dj].wait_send()

    sends2 = []
    for dj in range(1, N_DEV):
        d = dests[dj]
        rdma = pltpu.make_async_remote_copy(
            src_ref=red_ref,
            dst_ref=flat_ref.at[pl.ds(my * CHUNK, CHUNK), :],
            send_sem=s2_send.at[dj - 1],
            recv_sem=s2_recv.at[dj - 1],
            device_id=(d,),
            device_id_type=_MESH,
        )
        rdma.start()
        sends2.append(rdma)

    for k in range(N_DEV - 1):
        recv = pltpu.make_async_remote_copy(
            src_ref=red_ref, dst_ref=red_ref,
            send_sem=s2_send.at[k], recv_sem=s2_recv.at[k],
            device_id=(my,), device_id_type=_MESH,
        )
        recv.wait_recv()
    for r in sends2:
        r.wait_send()

    out_ref[0, :, :] = flat_ref[0:SQ, :].astype(jnp.float32)
    out_ref[1, :, :] = flat_ref[SQ:ROWS, :].astype(jnp.float32)


def kernel(x, Wq, K_ext, V_ext, Wo):
    i = lax.axis_index("i")
    k_sl = lax.dynamic_slice_in_dim(
        K_ext, i * HQ_LOCAL, HQ_LOCAL, axis=2).astype(jnp.bfloat16)
    v_sl = lax.dynamic_slice_in_dim(
        V_ext, i * HQ_LOCAL, HQ_LOCAL, axis=2).astype(jnp.bfloat16)

    return pl.pallas_call(
        _body,
        out_shape=jax.ShapeDtypeStruct((B, SQ, D_MODEL), jnp.float32),
        in_specs=[pl.BlockSpec(memory_space=pltpu.VMEM)] * 5,
        out_specs=pl.BlockSpec(memory_space=pltpu.VMEM),
        scratch_shapes=[
            pltpu.VMEM((ROWS, D_MODEL), jnp.bfloat16),
            pltpu.VMEM((B, SQ, D_LOCAL), jnp.bfloat16),
            pltpu.VMEM((N_DEV - 1, CHUNK, D_MODEL), jnp.bfloat16),
            pltpu.VMEM((CHUNK, D_MODEL), jnp.bfloat16),
            pltpu.VMEM((ROWS, D_MODEL), jnp.bfloat16),
            pltpu.SemaphoreType.DMA((N_DEV - 1,)),
            pltpu.SemaphoreType.DMA((N_DEV - 1,)),
            pltpu.SemaphoreType.DMA((N_DEV - 1,)),
            pltpu.SemaphoreType.DMA((N_DEV - 1,)),
        ],
        compiler_params=pltpu.CompilerParams(collective_id=0),
    )(x.astype(jnp.bfloat16), Wq.astype(jnp.bfloat16), k_sl, v_sl,
      Wo.astype(jnp.bfloat16))


# device time: 99506 ns/iter; 1.1530x vs baseline; 1.1530x over previous
import jax
import jax.numpy as jnp
from jax import lax
from jax.experimental import pallas as pl
from jax.experimental.pallas import tpu as pltpu

N_DEV = 16
B, SQ, SKV, DH = 2, 512, 512, 64
HQ_LOCAL = 8
D_LOCAL = HQ_LOCAL * DH
D_MODEL = 768
ROWS = B * SQ
CHUNK = ROWS // N_DEV

_MESH = pl.DeviceIdType.MESH


def _body(x_ref, wq_ref, k_ref, v_ref, wo_ref, out_ref,
          snd_ref, ctx_ref, a2a_ref, red_ref, flat_ref,
          s1_send, s1_recv, s2_send, s2_recv):
    my = lax.axis_index("i")

    barrier_sem = pltpu.get_barrier_semaphore()
    for dj in range(1, N_DEV):
        peer = lax.rem(my + dj, N_DEV)
        pl.semaphore_signal(barrier_sem, inc=1, device_id=(peer,),
                            device_id_type=_MESH)
    pl.semaphore_wait(barrier_sem, N_DEV - 1)

    dests = {}
    rdmas1 = {}
    for dj in range(1, N_DEV):
        d = lax.rem(my + dj, N_DEV)
        dests[dj] = d
        rdmas1[dj] = pltpu.make_async_remote_copy(
            src_ref=snd_ref.at[pl.ds(d * CHUNK, CHUNK), :],
            dst_ref=a2a_ref.at[dj - 1],
            send_sem=s1_send.at[dj - 1],
            recv_sem=s1_recv.at[dj - 1],
            device_id=(d,),
            device_id_type=_MESH,
        )

    qb_i = lax.broadcasted_iota(jnp.int32, (SQ, SKV), 0) // 64
    kb_i = lax.broadcasted_iota(jnp.int32, (SQ, SKV), 1) // 64
    bias = jnp.where((kb_i % 4) == (qb_i % 4), 0.0, -1e9)
    for b in range(B):
        q_b = jnp.dot(x_ref[b], wq_ref[...],
                      preferred_element_type=jnp.float32).astype(jnp.bfloat16)
        for h in range(HQ_LOCAL):
            q = q_b[:, h * DH:(h + 1) * DH]
            s = lax.dot_general(q, k_ref[b, h], (((1,), (1,)), ((), ())),
                                preferred_element_type=jnp.float32)
            w = jnp.exp(s * 0.125 + bias)
            w = w / jnp.sum(w, axis=1, keepdims=True)
            ctx = jnp.dot(w.astype(jnp.bfloat16), v_ref[b, h],
                          preferred_element_type=jnp.float32)
            ctx_ref[b, :, h * DH:(h + 1) * DH] = ctx.astype(jnp.bfloat16)
        proj = jnp.dot(ctx_ref[b], wo_ref[...],
                       preferred_element_type=jnp.float32)
        snd_ref[b * SQ:(b + 1) * SQ, :] = proj.astype(jnp.bfloat16)

        for dj in range(1, N_DEV):
            ready = (dests[dj] < 8) if b == 0 else (dests[dj] >= 8)
            rdma = rdmas1[dj]

            @pl.when(ready)
            def _(rdma=rdma):
                rdma.start()

    red = snd_ref[pl.ds(my * CHUNK, CHUNK), :].astype(jnp.float32)
    for k in range(N_DEV - 1):
        recv = pltpu.make_async_remote_copy(
            src_ref=a2a_ref.at[k], dst_ref=a2a_ref.at[k],
            send_sem=s1_send.at[k], recv_sem=s1_recv.at[k],
            device_id=(my,), device_id_type=_MESH,
        )
        recv.wait_recv()
        red = red + a2a_ref[k].astype(jnp.float32)
    red_ref[...] = red.astype(jnp.bfloat16)
    flat_ref[pl.ds(my * CHUNK, CHUNK), :] = red_ref[...]
    for dj in range(1, N_DEV):
        rdmas1[dj].wait_send()

    sends2 = []
    for dj in range(1, N_DEV):
        rdma = pltpu.make_async_remote_copy(
            src_ref=red_ref,
            dst_ref=flat_ref.at[pl.ds(my * CHUNK, CHUNK), :],
            send_sem=s2_send.at[dj - 1],
            recv_sem=s2_recv.at[dj - 1],
            device_id=(dests[dj],),
            device_id_type=_MESH,
        )
        rdma.start()
        sends2.append(rdma)

    for k in range(N_DEV - 1):
        recv = pltpu.make_async_remote_copy(
            src_ref=red_ref, dst_ref=red_ref,
            send_sem=s2_send.at[k], recv_sem=s2_recv.at[k],
            device_id=(my,), device_id_type=_MESH,
        )
        recv.wait_recv()
    for r in sends2:
        r.wait_send()

    out_ref[0, :, :] = flat_ref[0:SQ, :].astype(jnp.float32)
    out_ref[1, :, :] = flat_ref[SQ:ROWS, :].astype(jnp.float32)


def kernel(x, Wq, K_ext, V_ext, Wo):
    i = lax.axis_index("i")
    k_sl = lax.dynamic_slice_in_dim(K_ext, i * HQ_LOCAL, HQ_LOCAL, axis=2)
    v_sl = lax.dynamic_slice_in_dim(V_ext, i * HQ_LOCAL, HQ_LOCAL, axis=2)
    k_sl = jnp.transpose(k_sl, (0, 2, 1, 3)).astype(jnp.bfloat16)
    v_sl = jnp.transpose(v_sl, (0, 2, 1, 3)).astype(jnp.bfloat16)

    return pl.pallas_call(
        _body,
        out_shape=jax.ShapeDtypeStruct((B, SQ, D_MODEL), jnp.float32),
        in_specs=[pl.BlockSpec(memory_space=pltpu.VMEM)] * 5,
        out_specs=pl.BlockSpec(memory_space=pltpu.VMEM),
        scratch_shapes=[
            pltpu.VMEM((ROWS, D_MODEL), jnp.bfloat16),
            pltpu.VMEM((B, SQ, D_LOCAL), jnp.bfloat16),
            pltpu.VMEM((N_DEV - 1, CHUNK, D_MODEL), jnp.bfloat16),
            pltpu.VMEM((CHUNK, D_MODEL), jnp.bfloat16),
            pltpu.VMEM((ROWS, D_MODEL), jnp.bfloat16),
            pltpu.SemaphoreType.DMA((N_DEV - 1,)),
            pltpu.SemaphoreType.DMA((N_DEV - 1,)),
            pltpu.SemaphoreType.DMA((N_DEV - 1,)),
            pltpu.SemaphoreType.DMA((N_DEV - 1,)),
        ],
        compiler_params=pltpu.CompilerParams(collective_id=0),
    )(x.astype(jnp.bfloat16), Wq.astype(jnp.bfloat16), k_sl, v_sl,
      Wo.astype(jnp.bfloat16))
